# baseline (speedup 1.0000x reference)
import functools

import jax
import jax.numpy as jnp
from jax import lax
from jax.experimental import pallas as pl
from jax.experimental.pallas import tpu as pltpu
from jax.experimental.pallas import tpu_sc as plsc

N = 10000
E = 320000
D = 128
LANES = 16
NC, NS = 2, 16
NW = NC * NS
EPW = E // NW
C = 80
NCHUNK = EPW // C
ZCH = 80
NZCHUNK = N // ZCH


def _sc_message_kernel(h, packed_idx, e_emb):
  mesh = plsc.VectorSubcoreMesh(core_axis_name="c", subcore_axis_name="s")

  @functools.partial(
      pl.kernel,
      out_type=jax.ShapeDtypeStruct((NC, N, D), jnp.float32),
      mesh=mesh,
      scratch_types=[
          pltpu.VMEM((C,), jnp.int32),
          pltpu.VMEM((C,), jnp.int32),
          pltpu.VMEM((C,), jnp.int32),
          pltpu.VMEM((C,), jnp.int32),
          pltpu.VMEM((C, D), jnp.float32),
          pltpu.VMEM((C, D), jnp.float32),
          pltpu.VMEM((C, D), jnp.float32),
          pltpu.VMEM((C, D), jnp.float32),
          pltpu.VMEM_SHARED((N, D), jnp.float32),
          pltpu.SemaphoreType.DMA,
          pltpu.SemaphoreType.DMA,
          pltpu.SemaphoreType.DMA,
          pltpu.SemaphoreType.DMA,
          pltpu.SemaphoreType.DMA,
          pltpu.SemaphoreType.DMA,
      ],
  )
  def body(h_hbm, pk_hbm, e_hbm, out_hbm,
           src_v0, src_v1, dst_v0, dst_v1,
           rows_v0, rows_v1, e_v0, e_v1,
           agg_sp, si0, si1, sg0, sg1, se0, se1):
    cid = lax.axis_index("c")
    sid = lax.axis_index("s")
    wid = cid * NS + sid
    src_v = (src_v0, src_v1)
    dst_v = (dst_v0, dst_v1)
    rows_v = (rows_v0, rows_v1)
    e_v = (e_v0, e_v1)
    si = (si0, si1)
    sg = (sg0, sg1)
    se = (se0, se1)

    def zero_row(r, carry):
      for k in range(D // LANES):
        rows_v0[r, pl.ds(k * LANES, LANES)] = jnp.zeros((LANES,), jnp.float32)
      return carry
    lax.fori_loop(0, ZCH, zero_row, 0)
    def zero_chunk(t, carry):
      zc = sid + NS * t
      @pl.when(zc < NZCHUNK)
      def _():
        pltpu.sync_copy(rows_v0, agg_sp.at[pl.ds(zc * ZCH, ZCH)])
      return carry
    lax.fori_loop(0, (NZCHUNK + NS - 1) // NS, zero_chunk, 0)
    plsc.subcore_barrier()

    def idx_issue(j, b):
      base = wid * EPW + j * C
      pltpu.async_copy(pk_hbm.at[pl.ds(base, C)], src_v[b], si[b])

    def idx_wait(j, b):
      base = wid * EPW + j * C
      pltpu.make_async_copy(pk_hbm.at[pl.ds(base, C)], src_v[b], si[b]).wait()

    def unpack_idx(b):
      for g in range(C // LANES):
        sl = pl.ds(g * LANES, LANES)
        w = src_v[b][sl]
        dst_v[b][sl] = lax.bitwise_and(w, 16383)
        src_v[b][sl] = lax.shift_right_logical(w, 14)

    def ge_issue(j, b):
      pltpu.async_copy(h_hbm.at[src_v[b]], rows_v[b], sg[b])
      pltpu.async_copy(e_hbm.at[pl.ds(wid * EPW + j * C, C)], e_v[b], se[b])

    def ge_wait(j, b):
      pltpu.make_async_copy(h_hbm.at[src_v[b]], rows_v[b], sg[b]).wait()
      pltpu.make_async_copy(e_hbm.at[pl.ds(wid * EPW + j * C, C)],
                            e_v[b], se[b]).wait()

    def compute(b):
      rv, ev = rows_v[b], e_v[b]
      @plsc.parallel_loop(0, C, 1, unroll=2)
      def _(e):
        for k in range(D // LANES):
          sl = pl.ds(k * LANES, LANES)
          rv[e, sl] = jnp.maximum(rv[e, sl] + ev[e, sl], 0.0)

    def phase(j, b):
      ge_wait(j, b)
      @pl.when(j + 1 < NCHUNK)
      def _():
        idx_wait(j + 1, 1 - b)
        unpack_idx(1 - b)
        ge_issue(j + 1, 1 - b)
      compute(b)
      pltpu.sync_copy(rows_v[b], agg_sp.at[dst_v[b]], add=True)
      @pl.when(j + 2 < NCHUNK)
      def _():
        idx_issue(j + 2, b)

    idx_issue(0, 0)
    idx_wait(0, 0)
    unpack_idx(0)
    ge_issue(0, 0)
    idx_issue(1, 1)
    def pair(jj, carry):
      phase(2 * jj, 0)
      phase(2 * jj + 1, 1)
      return carry
    lax.fori_loop(0, NCHUNK // 2, pair, 0)
    phase(NCHUNK - 1, 0)
    plsc.subcore_barrier()

    def flush_chunk(t, carry):
      zc = sid + NS * t
      @pl.when(zc < NZCHUNK)
      def _():
        row0 = zc * ZCH
        pltpu.sync_copy(agg_sp.at[pl.ds(row0, ZCH)],
                        out_hbm.at[cid, pl.ds(row0, ZCH)])
      return carry
    lax.fori_loop(0, (NZCHUNK + NS - 1) // NS, flush_chunk, 0)

  return body(h, packed_idx, e_emb)


def _tc_atom_encoder(x, atom_W, atom_b):
  def body(x_ref, w_ref, b_ref, o_ref):
    o_ref[...] = jnp.dot(x_ref[...], w_ref[...],
                         preferred_element_type=jnp.float32) + b_ref[...]
  return pl.pallas_call(
      body, out_shape=jax.ShapeDtypeStruct((N, D), jnp.float32),
  )(x, atom_W, atom_b.reshape(1, D))


def _tc_edge_embed_one(edge_attr, edge_W_l, edge_b_l):
  K = edge_W_l.shape[0]
  BE = 8000

  def body(ea_ref, w_ref, b_ref, o_ref):
    o_ref[...] = jnp.dot(ea_ref[...], w_ref[...],
                         preferred_element_type=jnp.float32) + b_ref[...]

  return pl.pallas_call(
      body,
      grid=(E // BE,),
      in_specs=[
          pl.BlockSpec((BE, K), lambda i: (i, 0)),
          pl.BlockSpec((K, D), lambda i: (0, 0)),
          pl.BlockSpec((1, D), lambda i: (0, 0)),
      ],
      out_specs=pl.BlockSpec((BE, D), lambda i: (i, 0)),
      out_shape=jax.ShapeDtypeStruct((E, D), jnp.float32),
  )(edge_attr, edge_W_l, edge_b_l.reshape(1, D))


def _tc_update(h, agg_parts, W1, b1, W2, b2, eps, gamma, beta, relu_out):
  def body(h_ref, a_ref, w1_ref, b1_ref, w2_ref, b2_ref, eps_ref,
           g_ref, bt_ref, o_ref):
    z = (1.0 + eps_ref[0, 0]) * h_ref[...] + a_ref[0] + a_ref[1]
    t = jnp.maximum(jnp.dot(z, w1_ref[...],
                            preferred_element_type=jnp.float32)
                    + b1_ref[...], 0.0)
    t = jnp.dot(t, w2_ref[...], preferred_element_type=jnp.float32) + b2_ref[...]
    mean = jnp.mean(t, axis=0, keepdims=True)
    ctr = t - mean
    var = jnp.mean(ctr * ctr, axis=0, keepdims=True)
    out = g_ref[...] * ctr * lax.rsqrt(var + 1e-5) + bt_ref[...]
    if relu_out:
      out = jnp.maximum(out, 0.0)
    o_ref[...] = out

  return pl.pallas_call(
      body, out_shape=jax.ShapeDtypeStruct((N, D), jnp.float32),
  )(h, agg_parts, W1, b1.reshape(1, D), W2, b2.reshape(1, D),
    eps.reshape(1, 1), gamma.reshape(1, D), beta.reshape(1, D))


def kernel(x, edge_index, edge_attr, atom_W, atom_b, edge_W, edge_b,
           W1, b1, W2, b2, eps, gamma, beta):
  L = edge_W.shape[0]
  packed_idx = edge_index[0] * 16384 + edge_index[1]
  h = _tc_atom_encoder(x, atom_W, atom_b)
  e_cur = _tc_edge_embed_one(edge_attr, edge_W[0], edge_b[0])
  for l in range(L):
    agg_parts = _sc_message_kernel(h, packed_idx, e_cur)
    if l + 1 < L:
      e_cur = _tc_edge_embed_one(edge_attr, edge_W[l + 1], edge_b[l + 1])
    h = _tc_update(h, agg_parts, W1[l], b1[l], W2[l], b2[l],
                   eps[l], gamma[l], beta[l], relu_out=(l != L - 1))
  return h


# speedup vs baseline: 1.0157x; 1.0157x over previous
import functools

import jax
import jax.numpy as jnp
from jax import lax
from jax.experimental import pallas as pl
from jax.experimental.pallas import tpu as pltpu
from jax.experimental.pallas import tpu_sc as plsc

N = 10000
E = 320000
D = 128
LANES = 16
NC, NS = 2, 16
NW = NC * NS
EPW = E // NW
C = 80
NCHUNK = EPW // C
ZCH = 80
NZCHUNK = N // ZCH


def _sc_message_kernel(h, packed_idx, e_emb):
  mesh = plsc.VectorSubcoreMesh(core_axis_name="c", subcore_axis_name="s")

  @functools.partial(
      pl.kernel,
      out_type=jax.ShapeDtypeStruct((NC, N, D), jnp.float32),
      mesh=mesh,
      scratch_types=[
          pltpu.VMEM((C,), jnp.int32),
          pltpu.VMEM((C,), jnp.int32),
          pltpu.VMEM((C,), jnp.int32),
          pltpu.VMEM((C,), jnp.int32),
          pltpu.VMEM((C, D), jnp.float32),
          pltpu.VMEM((C, D), jnp.float32),
          pltpu.VMEM((C, D), jnp.float32),
          pltpu.VMEM((C, D), jnp.float32),
          pltpu.VMEM_SHARED((N, D), jnp.float32),
          pltpu.SemaphoreType.DMA,
          pltpu.SemaphoreType.DMA,
          pltpu.SemaphoreType.DMA,
          pltpu.SemaphoreType.DMA,
          pltpu.SemaphoreType.DMA,
          pltpu.SemaphoreType.DMA,
      ],
  )
  def body(h_hbm, pk_hbm, e_hbm, out_hbm,
           src_v0, src_v1, dst_v0, dst_v1,
           rows_v0, rows_v1, e_v0, e_v1,
           agg_sp, si0, si1, sg0, sg1, se0, se1):
    cid = lax.axis_index("c")
    sid = lax.axis_index("s")
    wid = cid * NS + sid
    src_v = (src_v0, src_v1)
    dst_v = (dst_v0, dst_v1)
    rows_v = (rows_v0, rows_v1)
    e_v = (e_v0, e_v1)
    si = (si0, si1)
    sg = (sg0, sg1)
    se = (se0, se1)

    def zero_row(r, carry):
      for k in range(D // LANES):
        rows_v0[r, pl.ds(k * LANES, LANES)] = jnp.zeros((LANES,), jnp.float32)
      return carry
    lax.fori_loop(0, ZCH, zero_row, 0)
    def zero_chunk(t, carry):
      zc = sid + NS * t
      @pl.when(zc < NZCHUNK)
      def _():
        pltpu.sync_copy(rows_v0, agg_sp.at[pl.ds(zc * ZCH, ZCH)])
      return carry
    lax.fori_loop(0, (NZCHUNK + NS - 1) // NS, zero_chunk, 0)
    plsc.subcore_barrier()

    def idx_issue(j, b):
      base = wid * EPW + j * C
      pltpu.async_copy(pk_hbm.at[pl.ds(base, C)], src_v[b], si[b])

    def idx_wait(j, b):
      base = wid * EPW + j * C
      pltpu.make_async_copy(pk_hbm.at[pl.ds(base, C)], src_v[b], si[b]).wait()

    def unpack_idx(b):
      for g in range(C // LANES):
        sl = pl.ds(g * LANES, LANES)
        w = src_v[b][sl]
        dst_v[b][sl] = lax.bitwise_and(w, 16383)
        src_v[b][sl] = lax.shift_right_logical(w, 14)

    def ge_issue(j, b):
      pltpu.async_copy(h_hbm.at[src_v[b]], rows_v[b], sg[b])
      pltpu.async_copy(e_hbm.at[pl.ds(wid * EPW + j * C, C)], e_v[b], se[b])

    def ge_wait(j, b):
      pltpu.make_async_copy(h_hbm.at[src_v[b]], rows_v[b], sg[b]).wait()
      pltpu.make_async_copy(e_hbm.at[pl.ds(wid * EPW + j * C, C)],
                            e_v[b], se[b]).wait()

    def compute(b):
      rv, ev = rows_v[b], e_v[b]
      @plsc.parallel_loop(0, C, 1, unroll=4)
      def _(e):
        for k in range(D // LANES):
          sl = pl.ds(k * LANES, LANES)
          rv[e, sl] = jnp.maximum(rv[e, sl] + ev[e, sl], 0.0)

    def phase(j, b):
      ge_wait(j, b)
      @pl.when(j + 1 < NCHUNK)
      def _():
        idx_wait(j + 1, 1 - b)
        unpack_idx(1 - b)
        ge_issue(j + 1, 1 - b)
      compute(b)
      pltpu.sync_copy(rows_v[b], agg_sp.at[dst_v[b]], add=True)
      @pl.when(j + 2 < NCHUNK)
      def _():
        idx_issue(j + 2, b)

    idx_issue(0, 0)
    idx_wait(0, 0)
    unpack_idx(0)
    ge_issue(0, 0)
    idx_issue(1, 1)
    def pair(jj, carry):
      phase(2 * jj, 0)
      phase(2 * jj + 1, 1)
      return carry
    lax.fori_loop(0, NCHUNK // 2, pair, 0)
    phase(NCHUNK - 1, 0)
    plsc.subcore_barrier()

    def flush_chunk(t, carry):
      zc = sid + NS * t
      @pl.when(zc < NZCHUNK)
      def _():
        row0 = zc * ZCH
        pltpu.sync_copy(agg_sp.at[pl.ds(row0, ZCH)],
                        out_hbm.at[cid, pl.ds(row0, ZCH)])
      return carry
    lax.fori_loop(0, (NZCHUNK + NS - 1) // NS, flush_chunk, 0)

  return body(h, packed_idx, e_emb)


def _tc_atom_encoder(x, atom_W, atom_b):
  def body(x_ref, w_ref, b_ref, o_ref):
    o_ref[...] = jnp.dot(x_ref[...], w_ref[...],
                         preferred_element_type=jnp.float32) + b_ref[...]
  return pl.pallas_call(
      body, out_shape=jax.ShapeDtypeStruct((N, D), jnp.float32),
  )(x, atom_W, atom_b.reshape(1, D))


def _tc_edge_embed(edge_attr, edge_W, edge_b):
  L, K, _ = edge_W.shape
  BE = 8000

  def body(ea_ref, w_ref, b_ref, o0_ref, o1_ref):
    ea = ea_ref[...]
    o0_ref[...] = jnp.dot(ea, w_ref[0],
                          preferred_element_type=jnp.float32) + b_ref[0]
    o1_ref[...] = jnp.dot(ea, w_ref[1],
                          preferred_element_type=jnp.float32) + b_ref[1]

  return pl.pallas_call(
      body,
      grid=(E // BE,),
      in_specs=[
          pl.BlockSpec((BE, K), lambda i: (i, 0)),
          pl.BlockSpec((L, K, D), lambda i: (0, 0, 0)),
          pl.BlockSpec((L, 1, D), lambda i: (0, 0, 0)),
      ],
      out_specs=[
          pl.BlockSpec((BE, D), lambda i: (i, 0)),
          pl.BlockSpec((BE, D), lambda i: (i, 0)),
      ],
      out_shape=[
          jax.ShapeDtypeStruct((E, D), jnp.float32),
          jax.ShapeDtypeStruct((E, D), jnp.float32),
      ],
  )(edge_attr, edge_W, edge_b.reshape(L, 1, D))


def _tc_update(h, agg_parts, W1, b1, W2, b2, eps, gamma, beta, relu_out):
  def body(h_ref, a_ref, w1_ref, b1_ref, w2_ref, b2_ref, eps_ref,
           g_ref, bt_ref, o_ref):
    z = (1.0 + eps_ref[0, 0]) * h_ref[...] + a_ref[0] + a_ref[1]
    t = jnp.maximum(jnp.dot(z, w1_ref[...],
                            preferred_element_type=jnp.float32)
                    + b1_ref[...], 0.0)
    t = jnp.dot(t, w2_ref[...], preferred_element_type=jnp.float32) + b2_ref[...]
    mean = jnp.mean(t, axis=0, keepdims=True)
    ctr = t - mean
    var = jnp.mean(ctr * ctr, axis=0, keepdims=True)
    out = g_ref[...] * ctr * lax.rsqrt(var + 1e-5) + bt_ref[...]
    if relu_out:
      out = jnp.maximum(out, 0.0)
    o_ref[...] = out

  return pl.pallas_call(
      body, out_shape=jax.ShapeDtypeStruct((N, D), jnp.float32),
  )(h, agg_parts, W1, b1.reshape(1, D), W2, b2.reshape(1, D),
    eps.reshape(1, 1), gamma.reshape(1, D), beta.reshape(1, D))


def kernel(x, edge_index, edge_attr, atom_W, atom_b, edge_W, edge_b,
           W1, b1, W2, b2, eps, gamma, beta):
  L = edge_W.shape[0]
  packed_idx = edge_index[0] * 16384 + edge_index[1]
  h = _tc_atom_encoder(x, atom_W, atom_b)
  e_emb = _tc_edge_embed(edge_attr, edge_W, edge_b)
  for l in range(L):
    agg_parts = _sc_message_kernel(h, packed_idx, e_emb[l])
    h = _tc_update(h, agg_parts, W1[l], b1[l], W2[l], b2[l],
                   eps[l], gamma[l], beta[l], relu_out=(l != L - 1))
  return h


# speedup vs baseline: 1.0461x; 1.0300x over previous
import functools

import jax
import jax.numpy as jnp
from jax import lax
from jax.experimental import pallas as pl
from jax.experimental.pallas import tpu as pltpu
from jax.experimental.pallas import tpu_sc as plsc

N = 10000
E = 320000
D = 128
LANES = 16
NC, NS = 2, 16
NW = NC * NS
EPW = E // NW
C = 80
NCHUNK = EPW // C
ZCH = 80
NZCHUNK = N // ZCH


def _sc_message_kernel(h, packed_idx, e_emb):
  mesh = plsc.VectorSubcoreMesh(core_axis_name="c", subcore_axis_name="s")

  @functools.partial(
      pl.kernel,
      out_type=jax.ShapeDtypeStruct((NC, N, D), jnp.float32),
      mesh=mesh,
      scratch_types=[
          pltpu.VMEM((C,), jnp.int32),
          pltpu.VMEM((C,), jnp.int32),
          pltpu.VMEM((C,), jnp.int32),
          pltpu.VMEM((C,), jnp.int32),
          pltpu.VMEM((C, D), jnp.float32),
          pltpu.VMEM((C, D), jnp.float32),
          pltpu.VMEM((C, D), jnp.float32),
          pltpu.VMEM((C, D), jnp.float32),
          pltpu.VMEM_SHARED((N, D), jnp.float32),
          pltpu.SemaphoreType.DMA,
          pltpu.SemaphoreType.DMA,
          pltpu.SemaphoreType.DMA,
          pltpu.SemaphoreType.DMA,
          pltpu.SemaphoreType.DMA,
          pltpu.SemaphoreType.DMA,
          pltpu.SemaphoreType.DMA,
          pltpu.SemaphoreType.DMA,
      ],
  )
  def body(h_hbm, pk_hbm, e_hbm, out_hbm,
           src_v0, src_v1, dst_v0, dst_v1,
           rows_v0, rows_v1, e_v0, e_v1,
           agg_sp, si0, si1, sg0, sg1, se0, se1, ss0, ss1):
    cid = lax.axis_index("c")
    sid = lax.axis_index("s")
    wid = cid * NS + sid
    src_v = (src_v0, src_v1)
    dst_v = (dst_v0, dst_v1)
    rows_v = (rows_v0, rows_v1)
    e_v = (e_v0, e_v1)
    si = (si0, si1)
    sg = (sg0, sg1)
    se = (se0, se1)
    ss = (ss0, ss1)

    def zero_row(r, carry):
      for k in range(D // LANES):
        rows_v0[r, pl.ds(k * LANES, LANES)] = jnp.zeros((LANES,), jnp.float32)
      return carry
    lax.fori_loop(0, ZCH, zero_row, 0)
    def zero_chunk(t, carry):
      zc = sid + NS * t
      @pl.when(zc < NZCHUNK)
      def _():
        pltpu.sync_copy(rows_v0, agg_sp.at[pl.ds(zc * ZCH, ZCH)])
      return carry
    lax.fori_loop(0, (NZCHUNK + NS - 1) // NS, zero_chunk, 0)
    plsc.subcore_barrier()

    def idx_issue(j, b):
      base = wid * EPW + j * C
      pltpu.async_copy(pk_hbm.at[pl.ds(base, C)], src_v[b], si[b])

    def idx_wait(j, b):
      base = wid * EPW + j * C
      pltpu.make_async_copy(pk_hbm.at[pl.ds(base, C)], src_v[b], si[b]).wait()

    def unpack_idx(b):
      for g in range(C // LANES):
        sl = pl.ds(g * LANES, LANES)
        w = src_v[b][sl]
        dst_v[b][sl] = lax.bitwise_and(w, 16383)
        src_v[b][sl] = lax.shift_right_logical(w, 14)

    def ge_issue(j, b):
      pltpu.async_copy(h_hbm.at[src_v[b]], rows_v[b], sg[b])
      pltpu.async_copy(e_hbm.at[pl.ds(wid * EPW + j * C, C)], e_v[b], se[b])

    def ge_wait(j, b):
      pltpu.make_async_copy(h_hbm.at[src_v[b]], rows_v[b], sg[b]).wait()
      pltpu.make_async_copy(e_hbm.at[pl.ds(wid * EPW + j * C, C)],
                            e_v[b], se[b]).wait()

    def compute(b):
      rv, ev = rows_v[b], e_v[b]
      @plsc.parallel_loop(0, C, 1, unroll=4)
      def _(e):
        for k in range(D // LANES):
          sl = pl.ds(k * LANES, LANES)
          rv[e, sl] = jnp.maximum(rv[e, sl] + ev[e, sl], 0.0)

    def scat_wait(b):
      pltpu.make_async_copy(rows_v[b], agg_sp.at[dst_v[b]], ss[b]).wait()

    def phase(j, b):
      ge_wait(j, b)
      @pl.when(j >= 1)
      def _():
        scat_wait(1 - b)
      @pl.when(j + 1 < NCHUNK)
      def _():
        idx_wait(j + 1, 1 - b)
        unpack_idx(1 - b)
        ge_issue(j + 1, 1 - b)
      compute(b)
      pltpu.async_copy(rows_v[b], agg_sp.at[dst_v[b]], ss[b], add=True)
      @pl.when(j + 2 < NCHUNK)
      def _():
        idx_issue(j + 2, b)

    idx_issue(0, 0)
    idx_wait(0, 0)
    unpack_idx(0)
    ge_issue(0, 0)
    idx_issue(1, 1)
    def pair(jj, carry):
      phase(2 * jj, 0)
      phase(2 * jj + 1, 1)
      return carry
    lax.fori_loop(0, NCHUNK // 2, pair, 0)
    phase(NCHUNK - 1, 0)
    scat_wait(0)
    plsc.subcore_barrier()

    def flush_chunk(t, carry):
      zc = sid + NS * t
      @pl.when(zc < NZCHUNK)
      def _():
        row0 = zc * ZCH
        pltpu.sync_copy(agg_sp.at[pl.ds(row0, ZCH)],
                        out_hbm.at[cid, pl.ds(row0, ZCH)])
      return carry
    lax.fori_loop(0, (NZCHUNK + NS - 1) // NS, flush_chunk, 0)

  return body(h, packed_idx, e_emb)


def _tc_atom_encoder(x, atom_W, atom_b):
  def body(x_ref, w_ref, b_ref, o_ref):
    o_ref[...] = jnp.dot(x_ref[...], w_ref[...],
                         preferred_element_type=jnp.float32) + b_ref[...]
  return pl.pallas_call(
      body, out_shape=jax.ShapeDtypeStruct((N, D), jnp.float32),
  )(x, atom_W, atom_b.reshape(1, D))


def _tc_edge_embed(edge_attr, edge_W, edge_b):
  L, K, _ = edge_W.shape
  BE = 8000

  def body(ea_ref, w_ref, b_ref, o0_ref, o1_ref):
    ea = ea_ref[...]
    o0_ref[...] = jnp.dot(ea, w_ref[0],
                          preferred_element_type=jnp.float32) + b_ref[0]
    o1_ref[...] = jnp.dot(ea, w_ref[1],
                          preferred_element_type=jnp.float32) + b_ref[1]

  return pl.pallas_call(
      body,
      grid=(E // BE,),
      in_specs=[
          pl.BlockSpec((BE, K), lambda i: (i, 0)),
          pl.BlockSpec((L, K, D), lambda i: (0, 0, 0)),
          pl.BlockSpec((L, 1, D), lambda i: (0, 0, 0)),
      ],
      out_specs=[
          pl.BlockSpec((BE, D), lambda i: (i, 0)),
          pl.BlockSpec((BE, D), lambda i: (i, 0)),
      ],
      out_shape=[
          jax.ShapeDtypeStruct((E, D), jnp.float32),
          jax.ShapeDtypeStruct((E, D), jnp.float32),
      ],
  )(edge_attr, edge_W, edge_b.reshape(L, 1, D))


def _tc_update(h, agg_parts, W1, b1, W2, b2, eps, gamma, beta, relu_out):
  def body(h_ref, a_ref, w1_ref, b1_ref, w2_ref, b2_ref, eps_ref,
           g_ref, bt_ref, o_ref):
    z = (1.0 + eps_ref[0, 0]) * h_ref[...] + a_ref[0] + a_ref[1]
    t = jnp.maximum(jnp.dot(z, w1_ref[...],
                            preferred_element_type=jnp.float32)
                    + b1_ref[...], 0.0)
    t = jnp.dot(t, w2_ref[...], preferred_element_type=jnp.float32) + b2_ref[...]
    mean = jnp.mean(t, axis=0, keepdims=True)
    ctr = t - mean
    var = jnp.mean(ctr * ctr, axis=0, keepdims=True)
    out = g_ref[...] * ctr * lax.rsqrt(var + 1e-5) + bt_ref[...]
    if relu_out:
      out = jnp.maximum(out, 0.0)
    o_ref[...] = out

  return pl.pallas_call(
      body, out_shape=jax.ShapeDtypeStruct((N, D), jnp.float32),
  )(h, agg_parts, W1, b1.reshape(1, D), W2, b2.reshape(1, D),
    eps.reshape(1, 1), gamma.reshape(1, D), beta.reshape(1, D))


def kernel(x, edge_index, edge_attr, atom_W, atom_b, edge_W, edge_b,
           W1, b1, W2, b2, eps, gamma, beta):
  L = edge_W.shape[0]
  packed_idx = edge_index[0] * 16384 + edge_index[1]
  h = _tc_atom_encoder(x, atom_W, atom_b)
  e_emb = _tc_edge_embed(edge_attr, edge_W, edge_b)
  for l in range(L):
    agg_parts = _sc_message_kernel(h, packed_idx, e_emb[l])
    h = _tc_update(h, agg_parts, W1[l], b1[l], W2[l], b2[l],
                   eps[l], gamma[l], beta[l], relu_out=(l != L - 1))
  return h
